# Initial kernel scaffold; baseline (speedup 1.0000x reference)
import jax
import jax.numpy as jnp
from jax.experimental import pallas as pl


def kernel(inputs, targets, cutpoints, Pi, C):
    raise NotImplementedError("write your pallas kernel here")


# speedup vs baseline: 5.0250x; 5.0250x over previous
import functools

import jax
import jax.numpy as jnp
from jax import lax
from jax.experimental import pallas as pl
from jax.experimental.pallas import tpu as pltpu
from jax.experimental.pallas import tpu_sc as plsc

_LANES = 16
_NW = 32


def _make_sc_call(n, K, Mc):
    per_w = n // _NW
    ch = per_w // _LANES
    mesh = plsc.VectorSubcoreMesh(core_axis_name="c", subcore_axis_name="s")

    @functools.partial(
        pl.kernel,
        mesh=mesh,
        out_type=jax.ShapeDtypeStruct((_NW * _LANES,), jnp.float32),
        compiler_params=pltpu.CompilerParams(needs_layout_passes=False),
        scratch_types=[
            pltpu.VMEM((per_w,), jnp.float32),
            pltpu.VMEM((per_w,), jnp.int32),
            pltpu.VMEM((128,), jnp.float32),
            pltpu.VMEM((K, _LANES), jnp.float32),
            pltpu.VMEM((_LANES,), jnp.float32),
        ],
    )
    def sc_body(x_hbm, t_hbm, tab_hbm, rbc_hbm, out_hbm, x_v, t_v, tab_v, rbc_v, acc_v):
        wid = lax.axis_index("c") * 16 + lax.axis_index("s")
        base = wid * per_w
        pltpu.sync_copy(x_hbm.at[pl.ds(base, per_w)], x_v)
        pltpu.sync_copy(t_hbm.at[pl.ds(base, per_w)], t_v)
        pltpu.sync_copy(tab_hbm, tab_v)
        pltpu.sync_copy(rbc_hbm, rbc_v)

        def body(i, acc):
            off = i * _LANES
            x = x_v[pl.ds(off, _LANES)]
            t = t_v[pl.ds(off, _LANES)]
            E = jnp.exp(x)
            ib = t * (K + 1)
            part = plsc.load_gather(tab_v, [ib + K])
            for j in range(K):
                w = plsc.load_gather(tab_v, [ib + j])
                part = part + w / (E + rbc_v[j, :])
            return acc + part

        acc = lax.fori_loop(0, ch, body, jnp.zeros((_LANES,), jnp.float32))
        acc_v[...] = acc
        pltpu.sync_copy(acc_v, out_hbm.at[pl.ds(wid * _LANES, _LANES)])

    return sc_body


def kernel(inputs, targets, cutpoints, Pi, C):
    Mc = C.shape[0]
    K = Mc - 1
    n = inputs.shape[0]

    r = jnp.exp(cutpoints.astype(jnp.float32))
    cost_diff = (C[:-1, :] - C[1:, :]).astype(jnp.float32)
    W2 = (cost_diff * Pi[None, :].astype(jnp.float32) * r[:, None]).T
    b = (Pi * C[-1, :]).astype(jnp.float32)
    table = jnp.concatenate([W2, b[:, None]], axis=1)
    tab_flat = jnp.zeros((128,), jnp.float32).at[: Mc * (K + 1)].set(
        table.reshape(-1))
    rbc = jnp.broadcast_to(r[:, None], (K, _LANES)).astype(jnp.float32)

    sc_call = _make_sc_call(n, K, Mc)
    partials = sc_call(inputs.astype(jnp.float32), targets.astype(jnp.int32),
                       tab_flat, rbc)
    return jnp.sum(partials)


# speedup vs baseline: 6.4122x; 1.2761x over previous
import functools

import jax
import jax.numpy as jnp
from jax import lax
from jax.experimental import pallas as pl
from jax.experimental.pallas import tpu as pltpu
from jax.experimental.pallas import tpu_sc as plsc

_LANES = 16
_NW = 32


def _tree_prod(vals):
    vals = list(vals)
    while len(vals) > 1:
        nxt = [a * b for a, b in zip(vals[0::2], vals[1::2])]
        if len(vals) % 2:
            nxt.append(vals[-1])
        vals = nxt
    return vals[0]


def _make_sc_call(n, K, Mc):
    per_w = n // _NW
    ch = per_w // _LANES
    mesh = plsc.VectorSubcoreMesh(core_axis_name="c", subcore_axis_name="s")

    def _const(v, dtype=jnp.int32):
        return jnp.full((_LANES,), v, dtype)

    @functools.partial(
        pl.kernel,
        mesh=mesh,
        out_type=jax.ShapeDtypeStruct((_NW * _LANES,), jnp.float32),
        compiler_params=pltpu.CompilerParams(needs_layout_passes=False),
        scratch_types=[
            pltpu.VMEM((per_w,), jnp.float32),
            pltpu.VMEM((per_w,), jnp.int32),
            pltpu.VMEM((Mc + 2, _LANES), jnp.float32),
            pltpu.VMEM((K + 1, _LANES), jnp.float32),
            pltpu.VMEM((_LANES,), jnp.float32),
            pltpu.VMEM((2 * _LANES,), jnp.float32),
            pltpu.VMEM((_LANES,), jnp.float32),
            pltpu.VMEM((_LANES,), jnp.float32),
            pltpu.SemaphoreType.DMA,
            pltpu.SemaphoreType.DMA,
        ],
    )
    def sc_body(x_hbm, t_hbm, sm_hbm, out_hbm,
                x_v, t_v, sm_v, tab_v, r_v, scr_v, po_v, acc_v, sem1, sem2):
        wid = lax.axis_index("c") * 16 + lax.axis_index("s")
        base = wid * per_w
        cp_x = pltpu.async_copy(x_hbm.at[pl.ds(base, per_w)], x_v, sem1)
        cp_t = pltpu.async_copy(t_hbm.at[pl.ds(base, per_w)], t_v, sem2)
        pltpu.sync_copy(sm_hbm, sm_v)

        zeros = jnp.zeros((_LANES,), jnp.float32)
        lane = lax.iota(jnp.int32, _LANES)

        cut = sm_v[Mc, :]
        pi = sm_v[Mc + 1, :]
        r = jnp.where(lane < K, jnp.exp(cut), zeros)
        r_v[...] = r
        rb = [plsc.load_gather(r_v, [_const(k)]) for k in range(K)]

        crows = [sm_v[j, :] for j in range(Mc)]
        wvec = [(crows[j] - crows[j + 1]) * pi * rb[j] for j in range(K)]
        bvec = pi * crows[Mc - 1]

        scr_v[pl.ds(0, _LANES)] = zeros
        scr_v[pl.ds(_LANES, _LANES)] = zeros

        def shift_down(v):
            scr_v[pl.ds(1, _LANES)] = v
            return scr_v[pl.ds(0, _LANES)]

        one0 = jnp.where(lane == 0, jnp.float32(1.0), zeros)
        q = one0
        for k in range(K):
            q = shift_down(q) + rb[k] * q
        po_v[...] = q
        qb = [plsc.load_gather(po_v, [_const(i)]) for i in range(K + 1)]
        qjb = []
        for j in range(K):
            v = one0
            for k in range(K):
                if k != j:
                    v = shift_down(v) + rb[k] * v
            po_v[...] = v
            qjb.append([plsc.load_gather(po_v, [_const(i)]) for i in range(K)])
        for k in range(K + 1):
            i = K - k
            row = bvec * qb[i]
            if i < K:
                for j in range(K):
                    row = row + wvec[j] * qjb[j][i]
            tab_v[k, :] = row

        cp_x.wait()
        cp_t.wait()

        r_vecs = rb

        def body(i, acc):
            off = i * _LANES
            x = x_v[pl.ds(off, _LANES)]
            t = t_v[pl.ds(off, _LANES)]
            E = jnp.minimum(jnp.exp(x), jnp.float32(1e4))
            qq = _tree_prod([E + rv for rv in r_vecs])
            p = plsc.load_gather(tab_v.at[0], [t])
            for k in range(1, K + 1):
                c = plsc.load_gather(tab_v.at[k], [t])
                p = p * E + c
            return acc + p / qq

        acc = lax.fori_loop(0, ch, body, zeros, unroll=4)
        acc_v[...] = acc
        pltpu.sync_copy(acc_v, out_hbm.at[pl.ds(wid * _LANES, _LANES)])

    return sc_body


def kernel(inputs, targets, cutpoints, Pi, C):
    Mc = C.shape[0]
    K = Mc - 1
    n = inputs.shape[0]

    sm = jnp.zeros((Mc + 2, _LANES), jnp.float32)
    sm = sm.at[:Mc, :Mc].set(C.astype(jnp.float32))
    sm = sm.at[Mc, :K].set(cutpoints.astype(jnp.float32))
    sm = sm.at[Mc + 1, :Mc].set(Pi.astype(jnp.float32))

    sc_call = _make_sc_call(n, K, Mc)
    partials = sc_call(inputs.astype(jnp.float32), targets.astype(jnp.int32),
                       sm)
    return jnp.sum(partials)
